# baseline (speedup 1.0000x reference)
import functools

import jax
import jax.numpy as jnp
from jax import lax
from jax.experimental import pallas as pl
from jax.experimental.pallas import tpu as pltpu
from jax.experimental.pallas import tpu_sc as plsc

EMB = 16
BATCH = 16384
NC = 2
NS = 16
NW = NC * NS
BPW = BATCH // NW
CH = 128


def _gather_body(u_hbm, i_hbm, uemb_hbm, iemb_hbm, ue_out, ie_out,
                 idx_u, idx_i, rows_u, rows_i, sem):
    wid = lax.axis_index("s") * NC + lax.axis_index("c")
    base = wid * BPW
    pltpu.sync_copy(u_hbm.at[pl.ds(base, BPW)], idx_u)
    pltpu.sync_copy(i_hbm.at[pl.ds(base, BPW)], idx_i)
    copies = []
    for c in range(BPW // CH):
        off = c * CH
        copies.append(pltpu.async_copy(
            uemb_hbm.at[idx_u.at[pl.ds(off, CH)]],
            rows_u.at[pl.ds(off, CH), :], sem))
        copies.append(pltpu.async_copy(
            iemb_hbm.at[idx_i.at[pl.ds(off, CH)]],
            rows_i.at[pl.ds(off, CH), :], sem))
    for cp in copies:
        cp.wait()
    pltpu.sync_copy(rows_u, ue_out.at[pl.ds(base, BPW)])
    pltpu.sync_copy(rows_i, ie_out.at[pl.ds(base, BPW)])


@jax.jit
def _sc_gather(u, i, user_emb, item_emb):
    mesh = plsc.VectorSubcoreMesh(core_axis_name="c", subcore_axis_name="s")
    f = functools.partial(
        pl.kernel,
        mesh=mesh,
        out_type=[
            jax.ShapeDtypeStruct((BATCH, EMB), jnp.float32),
            jax.ShapeDtypeStruct((BATCH, EMB), jnp.float32),
        ],
        scratch_types=[
            pltpu.VMEM((BPW,), jnp.int32),
            pltpu.VMEM((BPW,), jnp.int32),
            pltpu.VMEM((BPW, EMB), jnp.float32),
            pltpu.VMEM((BPW, EMB), jnp.float32),
            pltpu.SemaphoreType.DMA,
        ],
        compiler_params=pltpu.CompilerParams(use_tc_tiling_on_sc=False),
    )(_gather_body)
    return f(u, i, user_emb, item_emb)


def _mlp_body(ue_ref, ie_ref, w1a_ref, w1b_ref, b1_ref, w2_ref, b2_ref, out_ref):
    h = jnp.dot(ue_ref[...], w1a_ref[...], preferred_element_type=jnp.float32)
    h = h + jnp.dot(ie_ref[...], w1b_ref[...], preferred_element_type=jnp.float32)
    h = jnp.maximum(h + b1_ref[...], 0.0)
    o = jnp.sum(h * w2_ref[...], axis=1, keepdims=True) + b2_ref[...]
    out_ref[...] = 1.0 / (1.0 + jnp.exp(-o))


BM = 2048


@jax.jit
def _tc_mlp(ue, ie, w1a, w1b, b1, w2, b2):
    grid = (BATCH // BM,)
    return pl.pallas_call(
        _mlp_body,
        grid=grid,
        in_specs=[
            pl.BlockSpec((BM, EMB), lambda m: (m, 0)),
            pl.BlockSpec((BM, EMB), lambda m: (m, 0)),
            pl.BlockSpec((EMB, EMB), lambda m: (0, 0)),
            pl.BlockSpec((EMB, EMB), lambda m: (0, 0)),
            pl.BlockSpec((1, EMB), lambda m: (0, 0)),
            pl.BlockSpec((1, EMB), lambda m: (0, 0)),
            pl.BlockSpec((1, 1), lambda m: (0, 0)),
        ],
        out_specs=pl.BlockSpec((BM, 1), lambda m: (m, 0)),
        out_shape=jax.ShapeDtypeStruct((BATCH, 1), jnp.float32),
    )(ue, ie, w1a, w1b, b1, w2, b2)


def kernel(u, i, user_emb, item_emb, W1, b1, W2, b2):
    u = u.astype(jnp.int32)
    i = i.astype(jnp.int32)
    ue, ie = _sc_gather(u, i, user_emb, item_emb)
    w1a = W1[:EMB]
    w1b = W1[EMB:]
    b1r = b1.reshape(1, EMB)
    w2r = W2.reshape(1, EMB)
    b2r = b2.reshape(1, 1)
    return _tc_mlp(ue, ie, w1a, w1b, b1r, w2r, b2r)
